# baseline (speedup 1.0000x reference)
import functools

import jax
import jax.numpy as jnp
from jax import lax
from jax.experimental import pallas as pl
from jax.experimental.pallas import tpu as pltpu
from jax.experimental.pallas import tpu_sc as plsc

N_NODES = 10000
N_EDGES = 160000
D = 256
N_GRAPHS = 64

NC = 2
NS = 16
NW = NC * NS

ROWS_SMALL = 312
ROWS_BIG = 320
N_SMALL = 30
TRASH = ROWS_BIG
ACC_ROWS = ROWS_BIG + 6

BLK_E = 1280
NBLK = N_EDGES // BLK_E
NGRP = BLK_E // 16
NCH = BLK_E // 128
SUBK = 64
TRASH_OFF = N_EDGES + SUBK
CAPH = N_EDGES + SUBK + 128
PACK = 16384


def _segsum_body(y_hbm, src_hbm, dst_hbm, out_hbm, listC,
                 sblk, dblk, abuf, pbuf, gidx, pkb, neg, msgs, acc, sem):
    c = lax.axis_index("c")
    s = lax.axis_index("s")
    w = s * NC + c

    lo = jnp.where(w < N_SMALL, ROWS_SMALL * w,
                   ROWS_SMALL * N_SMALL + ROWS_BIG * (w - N_SMALL))
    hi = lo + jnp.where(w < N_SMALL, ROWS_SMALL, ROWS_BIG)
    tilebase = w * CAPH

    iota = lax.iota(jnp.int32, 16)
    fifteen = jnp.full((16,), 15, jnp.int32)
    neg16 = jnp.full((16,), -1, jnp.int32)
    zero16 = jnp.zeros((16,), jnp.float32)

    def _zrow(i, _):
        for d16 in range(D // 16):
            acc[i, pl.ds(d16 * 16, 16)] = zero16
        return 0

    lax.fori_loop(0, ROWS_BIG + 1, _zrow, 0)
    neg[pl.ds(0, 16)] = neg16

    def _psum16(v):
        for k in range(4):
            sh = 1 << k
            idx = jnp.maximum(iota - sh, 0)
            shifted = jnp.take(v, idx, axis=0, mode="wrap")
            v = v + jnp.where(iota >= sh, shifted, 0)
        return v

    def _block(b, offv):
        pltpu.sync_copy(src_hbm.at[pl.ds(b * BLK_E, BLK_E)], sblk)
        pltpu.sync_copy(dst_hbm.at[pl.ds(b * BLK_E, BLK_E)], dblk)

        def _grp(g, offv_):
            sv = sblk[pl.ds(g * 16, 16)]
            dv = dblk[pl.ds(g * 16, 16)]
            mine = (dv >= lo) & (dv < hi)
            mi = jnp.where(mine, 1, 0)
            incl = _psum16(mi)
            trash_addr = (tilebase + TRASH_OFF) + (iota + (g % 8) * 16)
            addr = jnp.where(mine, (offv_ + tilebase) + (incl - mi), trash_addr)
            pbuf[g // 8, pl.ds((g % 8) * 16, 16)] = sv * PACK + dv
            abuf[g // 8, pl.ds((g % 8) * 16, 16)] = addr
            return offv_ + jnp.take(incl, fifteen, axis=0, mode="wrap")

        offv = lax.fori_loop(0, NGRP, _grp, offv)
        for j in range(NCH):
            pltpu.async_copy(pbuf.at[j], listC.at[abuf.at[j]], sem).wait()
        return offv

    offv = lax.fori_loop(0, NBLK, _block, jnp.zeros((16,), jnp.int32))

    for k in range(SUBK // 16):
        pltpu.async_copy(
            neg, listC.at[(offv + tilebase) + (iota + 16 * k)], sem).wait()

    cnt = (offv + iota)[0]
    nsub = (cnt + SUBK - 1) // SUBK

    def _sub(j, _):
        pltpu.sync_copy(listC.at[pl.ds(tilebase + j * SUBK, SUBK)], pkb)
        for k in range(SUBK // 16):
            pk = pkb[pl.ds(k * 16, 16)]
            gidx[pl.ds(k * 16, 16)] = jnp.where(pk < 0, 0, pk >> 14)
        pltpu.async_copy(y_hbm.at[gidx], msgs, sem).wait()
        for k in range(SUBK // 16):
            pk = pkb[pl.ds(k * 16, 16)]
            for lane in range(16):
                p = pk[lane]
                rr = jnp.where(p < 0, TRASH, (p & (PACK - 1)) - lo)
                er = rr * 0 + (k * 16 + lane)
                for d16 in range(D // 16):
                    plsc.addupdate(acc.at[rr, pl.ds(d16 * 16, 16)],
                                   msgs[er, pl.ds(d16 * 16, 16)])
        return 0

    lax.fori_loop(0, nsub, _sub, 0)

    @pl.when(w < N_SMALL)
    def _():
        pltpu.sync_copy(acc.at[pl.ds(0, ROWS_SMALL)],
                        out_hbm.at[pl.ds(lo, ROWS_SMALL)])

    @pl.when(w >= N_SMALL)
    def _():
        pltpu.sync_copy(acc.at[pl.ds(0, ROWS_BIG)],
                        out_hbm.at[pl.ds(lo, ROWS_BIG)])


@functools.cache
def _make_segsum():
    return pl.kernel(
        _segsum_body,
        mesh=plsc.VectorSubcoreMesh(core_axis_name="c", subcore_axis_name="s"),
        out_type=(
            jax.ShapeDtypeStruct((N_NODES, D), jnp.float32),
            jax.ShapeDtypeStruct((NW * CAPH,), jnp.int32),
        ),
        scratch_types=[
            pltpu.VMEM((BLK_E,), jnp.int32),
            pltpu.VMEM((BLK_E,), jnp.int32),
            pltpu.VMEM((NCH, 128), jnp.int32),
            pltpu.VMEM((NCH, 128), jnp.int32),
            pltpu.VMEM((SUBK,), jnp.int32),
            pltpu.VMEM((SUBK,), jnp.int32),
            pltpu.VMEM((16,), jnp.int32),
            pltpu.VMEM((SUBK, D), jnp.float32),
            pltpu.VMEM((ACC_ROWS, D), jnp.float32),
            pltpu.SemaphoreType.DMA,
        ],
    )


def _segsum(y, src, dst):
    out, _ = _make_segsum()(y, src, dst)
    return out


ROWS_BLK = 1000
N_BLKS = N_NODES // ROWS_BLK


def _layer_body(s_ref, x_ref, wrel_ref, wroot_ref, b_ref, o_ref):
    o_ref[...] = jnp.maximum(
        jnp.dot(s_ref[...], wrel_ref[...], preferred_element_type=jnp.float32)
        + jnp.dot(x_ref[...], wroot_ref[...], preferred_element_type=jnp.float32)
        + b_ref[...],
        0.0,
    )


def _layer(S, X, W_rel, W_root, b2d):
    return pl.pallas_call(
        _layer_body,
        grid=(N_BLKS,),
        in_specs=[
            pl.BlockSpec((ROWS_BLK, D), lambda i: (i, 0)),
            pl.BlockSpec((ROWS_BLK, D), lambda i: (i, 0)),
            pl.BlockSpec((D, D), lambda i: (0, 0)),
            pl.BlockSpec((D, D), lambda i: (0, 0)),
            pl.BlockSpec((1, D), lambda i: (0, 0)),
        ],
        out_specs=pl.BlockSpec((ROWS_BLK, D), lambda i: (i, 0)),
        out_shape=jax.ShapeDtypeStruct((N_NODES, D), jnp.float32),
    )(S, X, W_rel, W_root, b2d)


def _poolfc_body(h_ref, b3_ref, wfc_ref, bfc_ref, o_ref, acc_ref):
    i = pl.program_id(0)

    @pl.when(i == 0)
    def _():
        acc_ref[...] = jnp.full((N_GRAPHS, D), -jnp.inf, jnp.float32)

    bids = b3_ref[0]
    h = h_ref[...]

    def _g(g, _):
        vals = jnp.where(bids == g, h, -jnp.inf)
        m = jnp.max(vals, axis=0, keepdims=True)
        acc_ref[pl.ds(g, 1), :] = jnp.maximum(acc_ref[pl.ds(g, 1), :], m)
        return 0

    lax.fori_loop(0, N_GRAPHS, _g, 0)

    @pl.when(i == pl.num_programs(0) - 1)
    def _():
        o_ref[...] = (
            jnp.dot(acc_ref[...], wfc_ref[...], preferred_element_type=jnp.float32)
            + bfc_ref[...]
        )


def _poolfc(h, batch3, Wfc, bfc2d):
    return pl.pallas_call(
        _poolfc_body,
        grid=(N_BLKS,),
        in_specs=[
            pl.BlockSpec((ROWS_BLK, D), lambda i: (i, 0)),
            pl.BlockSpec((1, ROWS_BLK, 1), lambda i: (i, 0, 0)),
            pl.BlockSpec((D, Wfc.shape[1]), lambda i: (0, 0)),
            pl.BlockSpec((1, Wfc.shape[1]), lambda i: (0, 0)),
        ],
        out_specs=pl.BlockSpec((N_GRAPHS, Wfc.shape[1]), lambda i: (0, 0)),
        out_shape=jax.ShapeDtypeStruct((N_GRAPHS, Wfc.shape[1]), jnp.float32),
        scratch_shapes=[pltpu.VMEM((N_GRAPHS, D), jnp.float32)],
    )(h, batch3, Wfc, bfc2d)


def kernel(x, edge_index, batch, W1_rel, b1, W1_root, W2_rel, b2, W2_root, Wfc, bfc):
    src = edge_index[0].astype(jnp.int32)
    dst = edge_index[1].astype(jnp.int32)
    batch3 = batch.astype(jnp.int32).reshape(N_BLKS, ROWS_BLK, 1)
    b1r = b1.reshape(1, -1)
    b2r = b2.reshape(1, -1)
    bfcr = bfc.reshape(1, -1)

    S1 = _segsum(x, src, dst)
    h1 = _layer(S1, x, W1_rel, W1_root, b1r)
    S2 = _segsum(h1, src, dst)
    h2 = _layer(S2, h1, W2_rel, W2_root, b2r)
    return _poolfc(h2, batch3, Wfc, bfcr)


# speedup vs baseline: 5.0248x; 5.0248x over previous
import functools

import jax
import jax.numpy as jnp
from jax import lax
from jax.experimental import pallas as pl
from jax.experimental.pallas import tpu as pltpu
from jax.experimental.pallas import tpu_sc as plsc

N_NODES = 10000
N_EDGES = 160000
D = 256
N_GRAPHS = 64

NC = 2
NS = 16
NW = NC * NS

ROWS_SMALL = 312
ROWS_BIG = 320
N_SMALL = 30
TRASH = ROWS_BIG
ACC_ROWS = ROWS_BIG + 6

BLK_E = 1280
NBLK = N_EDGES // BLK_E
NGRP = BLK_E // 16
NCH = BLK_E // 128
SUBK = 64
TRASH_OFF = N_EDGES + SUBK
TRASH_SLOTS = 512
CAPH = N_EDGES + SUBK + TRASH_SLOTS * 16
PACK = 16384


def _segsum_body(y_hbm, src_hbm, dst_hbm, out_hbm, listC,
                 sblk, dblk, abuf, pbuf, gidx, pkb, neg, msgs, acc, sem):
    c = lax.axis_index("c")
    s = lax.axis_index("s")
    w = s * NC + c

    lo = jnp.where(w < N_SMALL, ROWS_SMALL * w,
                   ROWS_SMALL * N_SMALL + ROWS_BIG * (w - N_SMALL))
    hi = lo + jnp.where(w < N_SMALL, ROWS_SMALL, ROWS_BIG)
    tilebase = w * CAPH

    iota = lax.iota(jnp.int32, 16)
    fifteen = jnp.full((16,), 15, jnp.int32)
    neg16 = jnp.full((16,), -1, jnp.int32)
    zero16 = jnp.zeros((16,), jnp.float32)

    def _zrow(i, _):
        for d16 in range(D // 16):
            acc[i, pl.ds(d16 * 16, 16)] = zero16
        return 0

    lax.fori_loop(0, ROWS_BIG + 1, _zrow, 0)
    neg[pl.ds(0, 16)] = neg16

    def _psum16(v):
        for k in range(4):
            sh = 1 << k
            idx = jnp.maximum(iota - sh, 0)
            shifted = jnp.take(v, idx, axis=0, mode="wrap")
            v = v + jnp.where(iota >= sh, shifted, 0)
        return v

    def _block(b, offv):
        pltpu.sync_copy(src_hbm.at[pl.ds(b * BLK_E, BLK_E)], sblk)
        pltpu.sync_copy(dst_hbm.at[pl.ds(b * BLK_E, BLK_E)], dblk)

        def _grp(g, offv_):
            sv = sblk[pl.ds(g * 16, 16)]
            dv = dblk[pl.ds(g * 16, 16)]
            mine = (dv >= lo) & (dv < hi)
            mi = jnp.where(mine, 1, 0)
            incl = _psum16(mi)
            tb = (tilebase + TRASH_OFF) + ((b * NGRP + g) % TRASH_SLOTS) * 16
            addr = jnp.where(mine, (offv_ + tilebase) + (incl - mi), tb + iota)
            pbuf[g // 8, pl.ds((g % 8) * 16, 16)] = sv * PACK + dv
            abuf[g // 8, pl.ds((g % 8) * 16, 16)] = addr
            return offv_ + jnp.take(incl, fifteen, axis=0, mode="wrap")

        offv = lax.fori_loop(0, NGRP, _grp, offv)
        copies = [pltpu.make_async_copy(pbuf.at[j], listC.at[abuf.at[j]], sem)
                  for j in range(NCH)]
        for cp in copies:
            cp.start()
        for cp in copies:
            cp.wait()
        return offv

    offv = lax.fori_loop(0, NBLK, _block, jnp.zeros((16,), jnp.int32))

    for k in range(SUBK // 16):
        pltpu.async_copy(
            neg, listC.at[(offv + tilebase) + (iota + 16 * k)], sem).wait()

    cnt = (offv + iota)[0]
    nsub = (cnt + SUBK - 1) // SUBK

    def _sub(j, _):
        pltpu.sync_copy(listC.at[pl.ds(tilebase + j * SUBK, SUBK)], pkb)
        for k in range(SUBK // 16):
            pk = pkb[pl.ds(k * 16, 16)]
            gidx[pl.ds(k * 16, 16)] = jnp.where(pk < 0, 0, pk >> 14)
        pltpu.async_copy(y_hbm.at[gidx], msgs, sem).wait()
        for k in range(SUBK // 16):
            pk = pkb[pl.ds(k * 16, 16)]
            for lane in range(16):
                p = pk[lane]
                rr = jnp.where(p < 0, TRASH, (p & (PACK - 1)) - lo)
                er = rr * 0 + (k * 16 + lane)
                for d16 in range(D // 16):
                    plsc.addupdate(acc.at[rr, pl.ds(d16 * 16, 16)],
                                   msgs[er, pl.ds(d16 * 16, 16)])
        return 0

    lax.fori_loop(0, nsub, _sub, 0)

    @pl.when(w < N_SMALL)
    def _():
        pltpu.sync_copy(acc.at[pl.ds(0, ROWS_SMALL)],
                        out_hbm.at[pl.ds(lo, ROWS_SMALL)])

    @pl.when(w >= N_SMALL)
    def _():
        pltpu.sync_copy(acc.at[pl.ds(0, ROWS_BIG)],
                        out_hbm.at[pl.ds(lo, ROWS_BIG)])


@functools.cache
def _make_segsum():
    return pl.kernel(
        _segsum_body,
        mesh=plsc.VectorSubcoreMesh(core_axis_name="c", subcore_axis_name="s"),
        out_type=(
            jax.ShapeDtypeStruct((N_NODES, D), jnp.float32),
            jax.ShapeDtypeStruct((NW * CAPH,), jnp.int32),
        ),
        scratch_types=[
            pltpu.VMEM((BLK_E,), jnp.int32),
            pltpu.VMEM((BLK_E,), jnp.int32),
            pltpu.VMEM((NCH, 128), jnp.int32),
            pltpu.VMEM((NCH, 128), jnp.int32),
            pltpu.VMEM((SUBK,), jnp.int32),
            pltpu.VMEM((SUBK,), jnp.int32),
            pltpu.VMEM((16,), jnp.int32),
            pltpu.VMEM((SUBK, D), jnp.float32),
            pltpu.VMEM((ACC_ROWS, D), jnp.float32),
            pltpu.SemaphoreType.DMA,
        ],
    )


def _segsum(y, src, dst):
    out, _ = _make_segsum()(y, src, dst)
    return out


ROWS_BLK = 1000
N_BLKS = N_NODES // ROWS_BLK


def _layer_body(s_ref, x_ref, wrel_ref, wroot_ref, b_ref, o_ref):
    o_ref[...] = jnp.maximum(
        jnp.dot(s_ref[...], wrel_ref[...], preferred_element_type=jnp.float32)
        + jnp.dot(x_ref[...], wroot_ref[...], preferred_element_type=jnp.float32)
        + b_ref[...],
        0.0,
    )


def _layer(S, X, W_rel, W_root, b2d):
    return pl.pallas_call(
        _layer_body,
        grid=(N_BLKS,),
        in_specs=[
            pl.BlockSpec((ROWS_BLK, D), lambda i: (i, 0)),
            pl.BlockSpec((ROWS_BLK, D), lambda i: (i, 0)),
            pl.BlockSpec((D, D), lambda i: (0, 0)),
            pl.BlockSpec((D, D), lambda i: (0, 0)),
            pl.BlockSpec((1, D), lambda i: (0, 0)),
        ],
        out_specs=pl.BlockSpec((ROWS_BLK, D), lambda i: (i, 0)),
        out_shape=jax.ShapeDtypeStruct((N_NODES, D), jnp.float32),
    )(S, X, W_rel, W_root, b2d)


def _poolfc_body(h_ref, b3_ref, wfc_ref, bfc_ref, o_ref, acc_ref):
    i = pl.program_id(0)

    @pl.when(i == 0)
    def _():
        acc_ref[...] = jnp.full((N_GRAPHS, D), -jnp.inf, jnp.float32)

    bids = b3_ref[0]
    h = h_ref[...]

    def _g(g, _):
        vals = jnp.where(bids == g, h, -jnp.inf)
        m = jnp.max(vals, axis=0, keepdims=True)
        acc_ref[pl.ds(g, 1), :] = jnp.maximum(acc_ref[pl.ds(g, 1), :], m)
        return 0

    lax.fori_loop(0, N_GRAPHS, _g, 0)

    @pl.when(i == pl.num_programs(0) - 1)
    def _():
        o_ref[...] = (
            jnp.dot(acc_ref[...], wfc_ref[...], preferred_element_type=jnp.float32)
            + bfc_ref[...]
        )


def _poolfc(h, batch3, Wfc, bfc2d):
    return pl.pallas_call(
        _poolfc_body,
        grid=(N_BLKS,),
        in_specs=[
            pl.BlockSpec((ROWS_BLK, D), lambda i: (i, 0)),
            pl.BlockSpec((1, ROWS_BLK, 1), lambda i: (i, 0, 0)),
            pl.BlockSpec((D, Wfc.shape[1]), lambda i: (0, 0)),
            pl.BlockSpec((1, Wfc.shape[1]), lambda i: (0, 0)),
        ],
        out_specs=pl.BlockSpec((N_GRAPHS, Wfc.shape[1]), lambda i: (0, 0)),
        out_shape=jax.ShapeDtypeStruct((N_GRAPHS, Wfc.shape[1]), jnp.float32),
        scratch_shapes=[pltpu.VMEM((N_GRAPHS, D), jnp.float32)],
    )(h, batch3, Wfc, bfc2d)


def kernel(x, edge_index, batch, W1_rel, b1, W1_root, W2_rel, b2, W2_root, Wfc, bfc):
    src = edge_index[0].astype(jnp.int32)
    dst = edge_index[1].astype(jnp.int32)
    batch3 = batch.astype(jnp.int32).reshape(N_BLKS, ROWS_BLK, 1)
    b1r = b1.reshape(1, -1)
    b2r = b2.reshape(1, -1)
    bfcr = bfc.reshape(1, -1)

    S1 = _segsum(x, src, dst)
    h1 = _layer(S1, x, W1_rel, W1_root, b1r)
    S2 = _segsum(h1, src, dst)
    h2 = _layer(S2, h1, W2_rel, W2_root, b2r)
    return _poolfc(h2, batch3, Wfc, bfcr)


# speedup vs baseline: 58.5370x; 11.6497x over previous
import functools

import jax
import jax.numpy as jnp
from jax import lax
from jax.experimental import pallas as pl
from jax.experimental.pallas import tpu as pltpu
from jax.experimental.pallas import tpu_sc as plsc

N_NODES = 10000
N_EDGES = 160000
D = 256
N_GRAPHS = 64

NC = 2
NS = 16
NW = NC * NS

ROWS_SMALL = 312
ROWS_BIG = 320
N_SMALL = 30
TRASH = ROWS_BIG
ACC_ROWS = ROWS_BIG + 6

BLK_E = 1280
NBLK = N_EDGES // BLK_E
NGRP = BLK_E // 16
SUBK = 64
CAP_W = 512
DRAIN_W = CAP_W - NGRP - 8
PACK = 16384


def _segsum_body(y_hbm, src_hbm, dst_hbm, out_hbm,
                 sblk, dblk, listV, gidx, msgs, acc, sem):
    c = lax.axis_index("c")
    s = lax.axis_index("s")
    w = s * NC + c

    lo = jnp.where(w < N_SMALL, ROWS_SMALL * w,
                   ROWS_SMALL * N_SMALL + ROWS_BIG * (w - N_SMALL))
    hi = lo + jnp.where(w < N_SMALL, ROWS_SMALL, ROWS_BIG)

    iota = lax.iota(jnp.int32, 16)
    fifteen = jnp.full((16,), 15, jnp.int32)
    neg16 = jnp.full((16,), -1, jnp.int32)
    zero16 = jnp.zeros((16,), jnp.float32)

    def _zrow(i, _):
        for d16 in range(D // 16):
            acc[i, pl.ds(d16 * 16, 16)] = zero16
        return 0

    lax.fori_loop(0, ROWS_BIG + 1, _zrow, 0)

    def _psum16(v):
        for k in range(4):
            sh = 1 << k
            idx = jnp.maximum(iota - sh, 0)
            shifted = jnp.take(v, idx, axis=0, mode="wrap")
            v = v + jnp.where(iota >= sh, shifted, 0)
        return v

    def _drain(listoff, p, pending):
        listV[pl.ds(listoff * 16, 16)] = jnp.where(iota < p, pending, -1)
        for k in range(1, 5):
            listV[pl.ds((listoff + k) * 16, 16)] = neg16
        cnt = listoff * 16 + p
        nsub = (cnt + SUBK - 1) // SUBK

        def _sub(j, _):
            for k in range(SUBK // 16):
                pk = listV[pl.ds(j * SUBK + k * 16, 16)]
                gidx[pl.ds(k * 16, 16)] = jnp.where(pk < 0, 0, pk >> 14)
            pltpu.async_copy(y_hbm.at[gidx], msgs, sem).wait()
            for k in range(SUBK // 16):
                pk = listV[pl.ds(j * SUBK + k * 16, 16)]
                for lane in range(16):
                    pe = pk[lane]
                    rr = jnp.where(pe < 0, TRASH, (pe & (PACK - 1)) - lo)
                    er = rr * 0 + (k * 16 + lane)
                    for d16 in range(D // 16):
                        plsc.addupdate(acc.at[rr, pl.ds(d16 * 16, 16)],
                                       msgs[er, pl.ds(d16 * 16, 16)])
            return 0

        lax.fori_loop(0, nsub, _sub, 0)

    def _block(b, carry):
        listoff, p, pending = carry
        pltpu.sync_copy(src_hbm.at[pl.ds(b * BLK_E, BLK_E)], sblk)
        pltpu.sync_copy(dst_hbm.at[pl.ds(b * BLK_E, BLK_E)], dblk)

        def _grp(g, carry_):
            listoff_, p_, pending_ = carry_
            sv = sblk[pl.ds(g * 16, 16)]
            dv = dblk[pl.ds(g * 16, 16)]
            mine = (dv >= lo) & (dv < hi)
            mi = jnp.where(mine, 1, 0)
            incl = _psum16(mi)
            cntg = incl[15]
            packed = jnp.where(mine, sv * PACK + dv, -1)
            losel = jnp.zeros((16,), jnp.int32)
            hisel = fifteen
            tgt = iota + 1
            for _bs in range(4):
                mid = (losel + hisel) >> 1
                vm = jnp.take(incl, mid, axis=0, mode="wrap")
                losel = jnp.where(vm < tgt, mid + 1, losel)
                hisel = jnp.where(vm < tgt, hisel, mid)
            comp = jnp.take(packed, losel, axis=0, mode="wrap")
            sh1 = jnp.take(comp, (iota - p_) & 15, axis=0, mode="wrap")
            pend2 = jnp.where(iota >= p_, jnp.where(iota < p_ + cntg, sh1, pending_), pending_)
            listV[pl.ds(listoff_ * 16, 16)] = pend2
            pnew = p_ + cntg
            sh2 = jnp.take(comp, (iota + 16 - p_) & 15, axis=0, mode="wrap")
            pend3 = jnp.where(iota < pnew - 16, sh2, pend2)
            p2 = jnp.where(pnew >= 16, pnew - 16, pnew)
            listoff2 = jnp.where(pnew >= 16, listoff_ + 1, listoff_)
            return (listoff2, p2, pend3)

        carry = lax.fori_loop(0, NGRP, _grp, (listoff, p, pending))
        listoff, p, pending = carry

        def _do_drain(cr):
            lo_, p_, pend_ = cr
            _drain(lo_, p_, pend_)
            return (jnp.int32(0), jnp.int32(0), pend_)

        return lax.cond(listoff >= DRAIN_W, _do_drain, lambda cr: cr, carry)

    carry = lax.fori_loop(
        0, NBLK, _block,
        (jnp.int32(0), jnp.int32(0), jnp.zeros((16,), jnp.int32)))
    listoff, p, pending = carry
    _drain(listoff, p, pending)

    @pl.when(w < N_SMALL)
    def _():
        pltpu.sync_copy(acc.at[pl.ds(0, ROWS_SMALL)],
                        out_hbm.at[pl.ds(lo, ROWS_SMALL)])

    @pl.when(w >= N_SMALL)
    def _():
        pltpu.sync_copy(acc.at[pl.ds(0, ROWS_BIG)],
                        out_hbm.at[pl.ds(lo, ROWS_BIG)])


@functools.cache
def _make_segsum():
    return pl.kernel(
        _segsum_body,
        mesh=plsc.VectorSubcoreMesh(core_axis_name="c", subcore_axis_name="s"),
        out_type=jax.ShapeDtypeStruct((N_NODES, D), jnp.float32),
        scratch_types=[
            pltpu.VMEM((BLK_E,), jnp.int32),
            pltpu.VMEM((BLK_E,), jnp.int32),
            pltpu.VMEM(((CAP_W + 8) * 16,), jnp.int32),
            pltpu.VMEM((SUBK,), jnp.int32),
            pltpu.VMEM((SUBK, D), jnp.float32),
            pltpu.VMEM((ACC_ROWS, D), jnp.float32),
            pltpu.SemaphoreType.DMA,
        ],
    )


def _segsum(y, src, dst):
    return _make_segsum()(y, src, dst)


ROWS_BLK = 1000
N_BLKS = N_NODES // ROWS_BLK


def _layer_body(s_ref, x_ref, wrel_ref, wroot_ref, b_ref, o_ref):
    o_ref[...] = jnp.maximum(
        jnp.dot(s_ref[...], wrel_ref[...], preferred_element_type=jnp.float32)
        + jnp.dot(x_ref[...], wroot_ref[...], preferred_element_type=jnp.float32)
        + b_ref[...],
        0.0,
    )


def _layer(S, X, W_rel, W_root, b2d):
    return pl.pallas_call(
        _layer_body,
        grid=(N_BLKS,),
        in_specs=[
            pl.BlockSpec((ROWS_BLK, D), lambda i: (i, 0)),
            pl.BlockSpec((ROWS_BLK, D), lambda i: (i, 0)),
            pl.BlockSpec((D, D), lambda i: (0, 0)),
            pl.BlockSpec((D, D), lambda i: (0, 0)),
            pl.BlockSpec((1, D), lambda i: (0, 0)),
        ],
        out_specs=pl.BlockSpec((ROWS_BLK, D), lambda i: (i, 0)),
        out_shape=jax.ShapeDtypeStruct((N_NODES, D), jnp.float32),
    )(S, X, W_rel, W_root, b2d)


def _poolfc_body(h_ref, b3_ref, wfc_ref, bfc_ref, o_ref, acc_ref):
    i = pl.program_id(0)

    @pl.when(i == 0)
    def _():
        acc_ref[...] = jnp.full((N_GRAPHS, D), -jnp.inf, jnp.float32)

    bids = b3_ref[0]
    h = h_ref[...]

    def _g(g, _):
        vals = jnp.where(bids == g, h, -jnp.inf)
        m = jnp.max(vals, axis=0, keepdims=True)
        acc_ref[pl.ds(g, 1), :] = jnp.maximum(acc_ref[pl.ds(g, 1), :], m)
        return 0

    lax.fori_loop(0, N_GRAPHS, _g, 0)

    @pl.when(i == pl.num_programs(0) - 1)
    def _():
        o_ref[...] = (
            jnp.dot(acc_ref[...], wfc_ref[...], preferred_element_type=jnp.float32)
            + bfc_ref[...]
        )


def _poolfc(h, batch3, Wfc, bfc2d):
    return pl.pallas_call(
        _poolfc_body,
        grid=(N_BLKS,),
        in_specs=[
            pl.BlockSpec((ROWS_BLK, D), lambda i: (i, 0)),
            pl.BlockSpec((1, ROWS_BLK, 1), lambda i: (i, 0, 0)),
            pl.BlockSpec((D, Wfc.shape[1]), lambda i: (0, 0)),
            pl.BlockSpec((1, Wfc.shape[1]), lambda i: (0, 0)),
        ],
        out_specs=pl.BlockSpec((N_GRAPHS, Wfc.shape[1]), lambda i: (0, 0)),
        out_shape=jax.ShapeDtypeStruct((N_GRAPHS, Wfc.shape[1]), jnp.float32),
        scratch_shapes=[pltpu.VMEM((N_GRAPHS, D), jnp.float32)],
    )(h, batch3, Wfc, bfc2d)


def kernel(x, edge_index, batch, W1_rel, b1, W1_root, W2_rel, b2, W2_root, Wfc, bfc):
    src = edge_index[0].astype(jnp.int32)
    dst = edge_index[1].astype(jnp.int32)
    batch3 = batch.astype(jnp.int32).reshape(N_BLKS, ROWS_BLK, 1)
    b1r = b1.reshape(1, -1)
    b2r = b2.reshape(1, -1)
    bfcr = bfc.reshape(1, -1)

    S1 = _segsum(x, src, dst)
    h1 = _layer(S1, x, W1_rel, W1_root, b1r)
    S2 = _segsum(h1, src, dst)
    h2 = _layer(S2, h1, W2_rel, W2_root, b2r)
    return _poolfc(h2, batch3, Wfc, bfcr)
